# Initial kernel scaffold; baseline (speedup 1.0000x reference)
import jax
import jax.numpy as jnp
from jax.experimental import pallas as pl


def kernel(x, position_ids, cos_sin_cache):
    raise NotImplementedError("write your pallas kernel here")


# speedup vs baseline: 1.0081x; 1.0081x over previous
import functools

import jax
import jax.numpy as jnp
from jax import lax
from jax.experimental import pallas as pl
from jax.experimental.pallas import tpu as pltpu
from jax.experimental.pallas import tpu_sc as plsc

MAX_POS = 32768
HEAD_DIM = 128
LANES = 16

NUM_CORES = 2
NUM_SUBCORES = 16
NW = NUM_CORES * NUM_SUBCORES

TOTAL = 32768
PER_W = TOTAL // NW
CHUNK = 128
NCH = PER_W // CHUNK


def _rotary_gather_body(cache2, idx, cos_out, sin_out,
                        idx_raw, idxc, idxs, cosbuf, sinbuf, sem):
    wid = lax.axis_index("s") * NUM_CORES + lax.axis_index("c")

    pltpu.sync_copy(idx.at[wid], idx_raw)

    for c in range(NCH):
        for j in range(CHUNK // LANES):
            v = idx_raw[c, pl.ds(j * LANES, LANES)]
            idxc[c, pl.ds(j * LANES, LANES)] = v + v
            idxs[c, pl.ds(j * LANES, LANES)] = v + v + 1

    for c in range(NCH):
        pltpu.async_copy(cache2.at[idxc.at[c]], cosbuf, sem).wait()
        pltpu.sync_copy(cosbuf, cos_out.at[wid, c])
        pltpu.async_copy(cache2.at[idxs.at[c]], sinbuf, sem).wait()
        pltpu.sync_copy(sinbuf, sin_out.at[wid, c])


@jax.jit
def _rotary_gather(cache2, idx):
    mesh = plsc.VectorSubcoreMesh(core_axis_name="c", subcore_axis_name="s")
    out_ty = jax.ShapeDtypeStruct((NW, NCH, CHUNK, HEAD_DIM), jnp.float32)
    run = pl.kernel(
        _rotary_gather_body,
        out_type=(out_ty, out_ty),
        mesh=mesh,
        scratch_types=[
            pltpu.VMEM((NCH, CHUNK), jnp.int32),
            pltpu.VMEM((NCH, CHUNK), jnp.int32),
            pltpu.VMEM((NCH, CHUNK), jnp.int32),
            pltpu.VMEM((CHUNK, HEAD_DIM), jnp.float32),
            pltpu.VMEM((CHUNK, HEAD_DIM), jnp.float32),
            pltpu.SemaphoreType.DMA,
        ],
    )
    return run(cache2, idx)


def kernel(x, position_ids, cos_sin_cache):
    del x
    bs, seq = position_ids.shape
    cache2 = cos_sin_cache.reshape(-1, HEAD_DIM)
    idx = position_ids.reshape(NW, NCH, CHUNK)
    cos, sin = _rotary_gather(cache2, idx)
    cos = cos.reshape(bs, seq, HEAD_DIM)
    sin = sin.reshape(bs, seq, HEAD_DIM)
    return (cos, sin)


# speedup vs baseline: 1.1567x; 1.1473x over previous
import functools

import jax
import jax.numpy as jnp
from jax import lax
from jax.experimental import pallas as pl
from jax.experimental.pallas import tpu as pltpu
from jax.experimental.pallas import tpu_sc as plsc

MAX_POS = 32768
HEAD_DIM = 128
LANES = 16

NUM_CORES = 2
NUM_SUBCORES = 16
NW = NUM_CORES * NUM_SUBCORES

TOTAL = 32768
PER_W = TOTAL // NW
CHUNK = 128
NCH = PER_W // CHUNK


NBUF = 3


def _rotary_gather_body(cache2, idx, cos_out, sin_out,
                        idx_raw, idxc, idxs, cosbuf, sinbuf, gsems, wsems):
    wid = lax.axis_index("s") * NUM_CORES + lax.axis_index("c")

    pltpu.sync_copy(idx.at[wid], idx_raw)

    for c in range(NCH):
        for j in range(CHUNK // LANES):
            v = idx_raw[c, pl.ds(j * LANES, LANES)]
            idxc[c, pl.ds(j * LANES, LANES)] = v + v
            idxs[c, pl.ds(j * LANES, LANES)] = v + v + 1

    gdesc, wdesc = {}, {}

    def fire_gather(c):
        b = c % NBUF
        gdesc[c] = (
            pltpu.async_copy(cache2.at[idxc.at[c]], cosbuf.at[b], gsems.at[b]),
            pltpu.async_copy(cache2.at[idxs.at[c]], sinbuf.at[b], gsems.at[b]),
        )

    fire_gather(0)
    fire_gather(1)
    for c in range(NCH):
        b = c % NBUF
        if c + 2 < NCH:
            if c - 1 >= 0:
                for d in wdesc[c - 1]:
                    d.wait()
            fire_gather(c + 2)
        for d in gdesc[c]:
            d.wait()
        wdesc[c] = (
            pltpu.async_copy(cosbuf.at[b], cos_out.at[wid, c], wsems.at[b]),
            pltpu.async_copy(sinbuf.at[b], sin_out.at[wid, c], wsems.at[b]),
        )
    for c in range(NCH - NBUF, NCH):
        for d in wdesc[c]:
            d.wait()


@jax.jit
def _rotary_gather(cache2, idx):
    mesh = plsc.VectorSubcoreMesh(core_axis_name="c", subcore_axis_name="s")
    out_ty = jax.ShapeDtypeStruct((NW, NCH, CHUNK, HEAD_DIM), jnp.float32)
    run = pl.kernel(
        _rotary_gather_body,
        out_type=(out_ty, out_ty),
        mesh=mesh,
        scratch_types=[
            pltpu.VMEM((NCH, CHUNK), jnp.int32),
            pltpu.VMEM((NCH, CHUNK), jnp.int32),
            pltpu.VMEM((NCH, CHUNK), jnp.int32),
            pltpu.VMEM((NBUF, CHUNK, HEAD_DIM), jnp.float32),
            pltpu.VMEM((NBUF, CHUNK, HEAD_DIM), jnp.float32),
            pltpu.SemaphoreType.DMA((NBUF,)),
            pltpu.SemaphoreType.DMA((NBUF,)),
        ],
    )
    return run(cache2, idx)


def kernel(x, position_ids, cos_sin_cache):
    del x
    bs, seq = position_ids.shape
    cache2 = cos_sin_cache.reshape(-1, HEAD_DIM)
    idx = position_ids.reshape(NW, NCH, CHUNK)
    cos, sin = _rotary_gather(cache2, idx)
    cos = cos.reshape(bs, seq, HEAD_DIM)
    sin = sin.reshape(bs, seq, HEAD_DIM)
    return (cos, sin)


# speedup vs baseline: 1.9519x; 1.6875x over previous
import functools

import jax
import jax.numpy as jnp
from jax import lax
from jax.experimental import pallas as pl
from jax.experimental.pallas import tpu as pltpu
from jax.experimental.pallas import tpu_sc as plsc

MAX_POS = 32768
HEAD_DIM = 128
CACHE_DIM = 2 * HEAD_DIM

NUM_CORES = 2
NUM_SUBCORES = 16
NW = NUM_CORES * NUM_SUBCORES

TOTAL = 32768
PER_W = TOTAL // NW
CHUNK = 128
NCH = PER_W // CHUNK


def _rotary_gather_body(cache, idx, cos_out, sin_out, idx_raw, rows, gsem, wsem):
    wid = lax.axis_index("s") * NUM_CORES + lax.axis_index("c")

    pltpu.sync_copy(idx.at[wid], idx_raw)

    pltpu.async_copy(cache.at[idx_raw.at[0]], rows.at[0], gsem.at[0])

    def step(c, carry):
        b = c % 2
        nb = (c + 1) % 2

        @pl.when(c + 1 < NCH)
        def _fire_next():
            @pl.when(c >= 1)
            def _drain_writes():
                pltpu.make_async_copy(
                    cache.at[pl.ds(0, CHUNK)], rows.at[nb], wsem.at[nb]
                ).wait()

            pltpu.async_copy(cache.at[idx_raw.at[c + 1]], rows.at[nb], gsem.at[nb])

        pltpu.make_async_copy(
            cache.at[pl.ds(0, CHUNK)], rows.at[b], gsem.at[b]
        ).wait()

        pltpu.async_copy(
            rows.at[b, :, pl.ds(0, HEAD_DIM)], cos_out.at[wid, c], wsem.at[b]
        )
        pltpu.async_copy(
            rows.at[b, :, pl.ds(HEAD_DIM, HEAD_DIM)], sin_out.at[wid, c], wsem.at[b]
        )
        return carry

    lax.fori_loop(0, NCH, step, 0)

    for b in range(2):
        pltpu.make_async_copy(
            cache.at[pl.ds(0, CHUNK)], rows.at[b], wsem.at[b]
        ).wait()


@jax.jit
def _rotary_gather(cache, idx):
    mesh = plsc.VectorSubcoreMesh(core_axis_name="c", subcore_axis_name="s")
    out_ty = jax.ShapeDtypeStruct((NW, NCH, CHUNK, HEAD_DIM), jnp.float32)
    run = pl.kernel(
        _rotary_gather_body,
        out_type=(out_ty, out_ty),
        mesh=mesh,
        scratch_types=[
            pltpu.VMEM((NCH, CHUNK), jnp.int32),
            pltpu.VMEM((2, CHUNK, CACHE_DIM), jnp.float32),
            pltpu.SemaphoreType.DMA((2,)),
            pltpu.SemaphoreType.DMA((2,)),
        ],
    )
    return run(cache, idx)


def kernel(x, position_ids, cos_sin_cache):
    del x
    bs, seq = position_ids.shape
    idx = position_ids.reshape(NW, NCH, CHUNK)
    cos, sin = _rotary_gather(cos_sin_cache, idx)
    cos = cos.reshape(bs, seq, HEAD_DIM)
    sin = sin.reshape(bs, seq, HEAD_DIM)
    return (cos, sin)


# speedup vs baseline: 1.9974x; 1.0233x over previous
import functools

import jax
import jax.numpy as jnp
from jax import lax
from jax.experimental import pallas as pl
from jax.experimental.pallas import tpu as pltpu
from jax.experimental.pallas import tpu_sc as plsc

MAX_POS = 32768
HEAD_DIM = 128
CACHE_DIM = 2 * HEAD_DIM

NUM_CORES = 2
NUM_SUBCORES = 16
NW = NUM_CORES * NUM_SUBCORES

BATCH = 4
SEQ = 8192
W_PER_B = NW // BATCH
PER_W = SEQ // W_PER_B
CHUNK = 128
NCH = PER_W // CHUNK
NBUF = 3


def _rotary_gather_body(cache, idx, cos_out, sin_out, idx_raw, rows, gsem, wsem):
    wid = lax.axis_index("s") * NUM_CORES + lax.axis_index("c")
    bi = wid // W_PER_B
    col = (wid % W_PER_B) * PER_W

    pltpu.sync_copy(idx.at[bi, pl.ds(col, PER_W)], idx_raw)

    pltpu.async_copy(cache.at[idx_raw.at[pl.ds(0, CHUNK)]], rows.at[0], gsem.at[0])
    pltpu.async_copy(
        cache.at[idx_raw.at[pl.ds(CHUNK, CHUNK)]], rows.at[1], gsem.at[1]
    )

    def step(c, carry):
        b = c % NBUF

        @pl.when(c + 2 < NCH)
        def _fire_next():
            nb = (c + 2) % NBUF
            @pl.when(c >= 1)
            def _drain_writes():
                pltpu.make_async_copy(
                    cache.at[pl.ds(0, CHUNK)], rows.at[nb], wsem.at[nb]
                ).wait()

            pltpu.async_copy(
                cache.at[idx_raw.at[pl.ds((c + 2) * CHUNK, CHUNK)]],
                rows.at[nb],
                gsem.at[nb],
            )

        pltpu.make_async_copy(
            cache.at[pl.ds(0, CHUNK)], rows.at[b], gsem.at[b]
        ).wait()

        dst = pl.ds(col + c * CHUNK, CHUNK)
        pltpu.async_copy(
            rows.at[b, :, pl.ds(0, HEAD_DIM)], cos_out.at[bi, dst], wsem.at[b]
        )
        pltpu.async_copy(
            rows.at[b, :, pl.ds(HEAD_DIM, HEAD_DIM)], sin_out.at[bi, dst], wsem.at[b]
        )
        return carry

    lax.fori_loop(0, NCH, step, 0)

    for b in range(NBUF):
        pltpu.make_async_copy(
            cache.at[pl.ds(0, CHUNK)], rows.at[b], wsem.at[b]
        ).wait()


@jax.jit
def _rotary_gather(cache, idx):
    mesh = plsc.VectorSubcoreMesh(core_axis_name="c", subcore_axis_name="s")
    out_ty = jax.ShapeDtypeStruct((BATCH, SEQ, HEAD_DIM), jnp.float32)
    run = pl.kernel(
        _rotary_gather_body,
        out_type=(out_ty, out_ty),
        mesh=mesh,
        scratch_types=[
            pltpu.VMEM((PER_W,), jnp.int32),
            pltpu.VMEM((NBUF, CHUNK, CACHE_DIM), jnp.float32),
            pltpu.SemaphoreType.DMA((NBUF,)),
            pltpu.SemaphoreType.DMA((NBUF,)),
        ],
    )
    return run(cache, idx)


def kernel(x, position_ids, cos_sin_cache):
    del x
    return _rotary_gather(cos_sin_cache, position_ids)
